# baseline (speedup 1.0000x reference)
import functools

import jax
import jax.numpy as jnp
from jax import lax
from jax.experimental import pallas as pl
from jax.experimental.pallas import tpu as pltpu
from jax.experimental.pallas import tpu_sc as plsc

_NC = 2
_NS = 16
_NW = _NC * _NS

_B, _T, _D = 4, 8192, 1024
_K = _T // 2
_ROWS = _B * _K
_RPW = _ROWS // _NW
_CH = 32
_NCHUNK = _RPW // _CH


_NV = _T // 16
_RADIX_BINS = 256 * 16


_NG = 4
_NVG = _NV // _NG


def _sort_body(scores_hbm, idx_hbm, sc_v, k0, v0, k1, v1, bins0, bins1, bins2,
               bins3, outb):
    wid = lax.axis_index("s") * _NC + lax.axis_index("c")
    binsg = (bins0, bins1, bins2, bins3)

    @pl.when(wid < _B)
    def _():
        r = wid
        pltpu.sync_copy(scores_hbm.at[r], sc_v)
        lane = lax.iota(jnp.int32, 16)
        ones = jnp.ones((16,), jnp.int32)
        zeros = jnp.zeros((16,), jnp.int32)

        def build(i, c):
            f = sc_v[pl.ds(i * 16, 16)]
            u = lax.bitcast_convert_type(f, jnp.int32)
            m = lax.shift_right_arithmetic(u, 31)
            key = u ^ (jnp.bitwise_not(m) & jnp.int32(0x7FFFFFFF))
            p = i * 16 + lane
            a = (p & (_NV - 1)) * 16 + lax.shift_right_logical(p, 9)
            plsc.store_scatter(k0, [a], key)
            plsc.store_scatter(v0, [a], p)
            return c

        lax.fori_loop(0, _NV, build, 0, unroll=4)

        bufs = ((k0, v0, k1, v1), (k1, v1, k0, v0))
        for pas in range(4):
            kin, vin, kout, vout = bufs[pas % 2]
            sh = 8 * pas

            def zb(i, c):
                for g in range(_NG):
                    binsg[g][pl.ds(i * 16, 16)] = zeros
                return c

            lax.fori_loop(0, _RADIX_BINS // 16, zb, 0, unroll=2)

            def hist(i, c):
                for g in range(_NG):
                    k = kin[pl.ds((g * _NVG + i) * 16, 16)]
                    d = (lax.shift_right_logical(k, sh) & jnp.int32(0xFF)) * 16 + lane
                    plsc.addupdate_scatter(binsg[g], [d], ones)
                return c

            lax.fori_loop(0, _NVG, hist, 0, unroll=2)

            def scan(i, run):
                sl = pl.ds(i * 16, 16)
                b = [binsg[g][sl] for g in range(_NG)]
                tot = b[0] + b[1] + b[2] + b[3]
                excl = (plsc.cumsum(tot) - tot) + run
                for g in range(_NG):
                    binsg[g][sl] = excl
                    excl = excl + b[g]
                return run + jnp.sum(tot)

            lax.fori_loop(0, _RADIX_BINS // 16, scan, jnp.int32(0))

            if pas < 3:

                def perm(i, c):
                    for g in range(_NG):
                        sl = pl.ds((g * _NVG + i) * 16, 16)
                        k = kin[sl]
                        val = vin[sl]
                        d = (lax.shift_right_logical(k, sh) & jnp.int32(0xFF)) * 16 + lane
                        q = plsc.load_gather(binsg[g], [d])
                        a = (q & (_NV - 1)) * 16 + lax.shift_right_logical(q, 9)
                        plsc.store_scatter(kout, [a], k)
                        plsc.store_scatter(vout, [a], val)
                        plsc.store_scatter(binsg[g], [d], q + 1)
                    return c

            else:
                def perm(i, c):
                    for g in range(_NG):
                        sl = pl.ds((g * _NVG + i) * 16, 16)
                        k = kin[sl]
                        val = vin[sl]
                        d = (lax.shift_right_logical(k, sh) & jnp.int32(0xFF)) * 16 + lane
                        q = plsc.load_gather(binsg[g], [d])
                        plsc.store_scatter(outb, [q], val, mask=q < _K)
                        plsc.store_scatter(binsg[g], [d], q + 1)
                    return c

            lax.fori_loop(0, _NVG, perm, 0, unroll=2)

        pltpu.sync_copy(outb, idx_hbm.at[r])


def _sc_sort(rand_scores):
    mesh = plsc.VectorSubcoreMesh(
        core_axis_name="c", subcore_axis_name="s", num_cores=_NC, num_subcores=_NS
    )
    return pl.kernel(
        _sort_body,
        out_type=jax.ShapeDtypeStruct((_B, _K), jnp.int32),
        mesh=mesh,
        compiler_params=pltpu.CompilerParams(
            use_tc_tiling_on_sc=False, needs_layout_passes=False
        ),
        scratch_types=[
            pltpu.VMEM((_T,), jnp.float32),
            pltpu.VMEM((_T,), jnp.int32),
            pltpu.VMEM((_T,), jnp.int32),
            pltpu.VMEM((_T,), jnp.int32),
            pltpu.VMEM((_T,), jnp.int32),
            pltpu.VMEM((_RADIX_BINS,), jnp.int32),
            pltpu.VMEM((_RADIX_BINS,), jnp.int32),
            pltpu.VMEM((_RADIX_BINS,), jnp.int32),
            pltpu.VMEM((_RADIX_BINS,), jnp.int32),
            pltpu.VMEM((_K,), jnp.int32),
        ],
    )(rand_scores)


def _gather_body(x_hbm, idx_hbm, out_hbm, idx_v, buf0, buf1, sem0, sem1):
    wid = lax.axis_index("s") * _NC + lax.axis_index("c")
    base = wid * _RPW
    pltpu.sync_copy(idx_hbm.at[wid], idx_v)

    bufs = (buf0, buf1)
    sems = (sem0, sem1)
    d0 = pltpu.async_copy(x_hbm.at[idx_v.at[0]], bufs[0], sems[0])
    descs = [d0, None]
    for c in range(_NCHUNK):
        descs[c % 2].wait()
        if c + 1 < _NCHUNK:
            descs[(c + 1) % 2] = pltpu.async_copy(
                x_hbm.at[idx_v.at[c + 1]], bufs[(c + 1) % 2], sems[(c + 1) % 2]
            )
        pltpu.sync_copy(bufs[c % 2], out_hbm.at[pl.ds(base + c * _CH, _CH)])


def _sc_gather(x_flat, idx_chunked):
    mesh = plsc.VectorSubcoreMesh(
        core_axis_name="c", subcore_axis_name="s", num_cores=_NC, num_subcores=_NS
    )
    return pl.kernel(
        _gather_body,
        out_type=jax.ShapeDtypeStruct((_ROWS, _D), jnp.float32),
        mesh=mesh,
        scratch_types=[
            pltpu.VMEM((_NCHUNK, _CH), jnp.int32),
            pltpu.VMEM((_CH, _D), jnp.float32),
            pltpu.VMEM((_CH, _D), jnp.float32),
            pltpu.SemaphoreType.DMA,
            pltpu.SemaphoreType.DMA,
        ],
    )(x_flat, idx_chunked)


def kernel(x, rand_scores):
    B, T, D = x.shape
    num_keep = _K
    token_indices_keep = _sc_sort(rand_scores)
    gidx = token_indices_keep + (jnp.arange(B, dtype=jnp.int32) * T)[:, None]
    gidx = gidx.reshape(_NW, _NCHUNK, _CH)
    out = _sc_gather(x.reshape(B * T, D), gidx)
    return (out.reshape(B, num_keep, D), token_indices_keep)


# speedup vs baseline: 1.4100x; 1.4100x over previous
import functools

import jax
import jax.numpy as jnp
from jax import lax
from jax.experimental import pallas as pl
from jax.experimental.pallas import tpu as pltpu
from jax.experimental.pallas import tpu_sc as plsc

_NC = 2
_NS = 16
_NW = _NC * _NS

_B, _T, _D = 4, 8192, 1024
_K = _T // 2
_ROWS = _B * _K
_RPW = _ROWS // _NW
_CH = 32
_NCHUNK = _RPW // _CH


_NV = _T // 16
_RADIX_BINS = 256 * 16


_NG = 4
_NVG = _NV // _NG


def _sort_body(scores_hbm, idx_hbm, sc_v, k0, v0, k1, v1, bins0, bins1, bins2,
               bins3, outb):
    wid = lax.axis_index("s") * _NC + lax.axis_index("c")
    binsg = (bins0, bins1, bins2, bins3)

    @pl.when(wid < _B)
    def _():
        r = wid
        pltpu.sync_copy(scores_hbm.at[r], sc_v)
        lane = lax.iota(jnp.int32, 16)
        ones = jnp.ones((16,), jnp.int32)
        zeros = jnp.zeros((16,), jnp.int32)

        def build(i, c):
            f = sc_v[pl.ds(i * 16, 16)]
            u = lax.bitcast_convert_type(f, jnp.int32)
            m = lax.shift_right_arithmetic(u, 31)
            key = u ^ (jnp.bitwise_not(m) & jnp.int32(0x7FFFFFFF))
            p = i * 16 + lane
            a = (p & (_NV - 1)) * 16 + lax.shift_right_logical(p, 9)
            plsc.store_scatter(k0, [a], key)
            plsc.store_scatter(v0, [a], p)
            return c

        lax.fori_loop(0, _NV, build, 0, unroll=4)

        bufs = ((k0, v0, k1, v1), (k1, v1, k0, v0))
        for pas in range(4):
            kin, vin, kout, vout = bufs[pas % 2]
            sh = 8 * pas

            def zb(i, c):
                for g in range(_NG):
                    binsg[g][pl.ds(i * 16, 16)] = zeros
                return c

            lax.fori_loop(0, _RADIX_BINS // 16, zb, 0, unroll=2)

            def hist(i, c):
                for g in range(_NG):
                    k = kin[pl.ds((g * _NVG + i) * 16, 16)]
                    d = (lax.shift_right_logical(k, sh) & jnp.int32(0xFF)) * 16 + lane
                    plsc.addupdate_scatter(binsg[g], [d], ones)
                return c

            lax.fori_loop(0, _NVG, hist, 0, unroll=2)

            def scan(i, run):
                sl = pl.ds(i * 16, 16)
                b = [binsg[g][sl] for g in range(_NG)]
                tot = b[0] + b[1] + b[2] + b[3]
                excl = (plsc.cumsum(tot) - tot) + run
                for g in range(_NG):
                    binsg[g][sl] = excl
                    excl = excl + b[g]
                return run + jnp.sum(tot)

            lax.fori_loop(0, _RADIX_BINS // 16, scan, jnp.int32(0))

            if pas < 3:

                def perm(i, c):
                    for g in range(_NG):
                        sl = pl.ds((g * _NVG + i) * 16, 16)
                        k = kin[sl]
                        val = vin[sl]
                        d = (lax.shift_right_logical(k, sh) & jnp.int32(0xFF)) * 16 + lane
                        q = plsc.load_gather(binsg[g], [d])
                        a = (q & (_NV - 1)) * 16 + lax.shift_right_logical(q, 9)
                        plsc.store_scatter(kout, [a], k)
                        plsc.store_scatter(vout, [a], val)
                        plsc.store_scatter(binsg[g], [d], q + 1)
                    return c

            else:
                def perm(i, c):
                    for g in range(_NG):
                        sl = pl.ds((g * _NVG + i) * 16, 16)
                        k = kin[sl]
                        val = vin[sl]
                        d = (lax.shift_right_logical(k, sh) & jnp.int32(0xFF)) * 16 + lane
                        q = plsc.load_gather(binsg[g], [d])
                        plsc.store_scatter(outb, [q], val, mask=q < _K)
                        plsc.store_scatter(binsg[g], [d], q + 1)
                    return c

            lax.fori_loop(0, _NVG, perm, 0, unroll=2)

        pltpu.sync_copy(outb, idx_hbm.at[r])


def _sc_sort(rand_scores):
    mesh = plsc.VectorSubcoreMesh(
        core_axis_name="c", subcore_axis_name="s", num_cores=_NC, num_subcores=_NS
    )
    return pl.kernel(
        _sort_body,
        out_type=jax.ShapeDtypeStruct((_B, _K), jnp.int32),
        mesh=mesh,
        compiler_params=pltpu.CompilerParams(
            use_tc_tiling_on_sc=False, needs_layout_passes=False
        ),
        scratch_types=[
            pltpu.VMEM((_T,), jnp.float32),
            pltpu.VMEM((_T,), jnp.int32),
            pltpu.VMEM((_T,), jnp.int32),
            pltpu.VMEM((_T,), jnp.int32),
            pltpu.VMEM((_T,), jnp.int32),
            pltpu.VMEM((_RADIX_BINS,), jnp.int32),
            pltpu.VMEM((_RADIX_BINS,), jnp.int32),
            pltpu.VMEM((_RADIX_BINS,), jnp.int32),
            pltpu.VMEM((_RADIX_BINS,), jnp.int32),
            pltpu.VMEM((_K,), jnp.int32),
        ],
    )(rand_scores)


def _tc_sort_body(s_ref, o_ref):
    f = s_ref[...]
    u = lax.bitcast_convert_type(f, jnp.int32)
    u = jnp.where(u == jnp.int32(-2147483648), jnp.int32(0), u)
    m = lax.shift_right_arithmetic(u, 31)
    key = u ^ (m & jnp.int32(0x7FFFFFFF))
    rsub = lax.broadcasted_iota(jnp.int32, (64, 128), 0)
    lan = lax.broadcasted_iota(jnp.int32, (64, 128), 1)
    idx = rsub * 128 + lan
    K, V = key, idx
    for kk in [2 << t for t in range(13)]:
        mk = (
            (lan & kk) == 0 if kk < 128 else (rsub & (kk // 128)) == 0
        )
        jj = kk // 2
        while jj >= 1:
            if jj < 128:
                mj = (lan & jj) == 0
                Kp = jnp.where(mj, pltpu.roll(K, 128 - jj, 1), pltpu.roll(K, jj, 1))
                Vp = jnp.where(mj, pltpu.roll(V, 128 - jj, 1), pltpu.roll(V, jj, 1))
            else:
                js = jj // 128
                mj = (rsub & js) == 0
                Kp = jnp.where(mj, pltpu.roll(K, 64 - js, 0), pltpu.roll(K, js, 0))
                Vp = jnp.where(mj, pltpu.roll(V, 64 - js, 0), pltpu.roll(V, js, 0))
            self_first = (K > Kp) | ((K == Kp) & (V < Vp))
            keep = jnp.logical_xor(self_first, jnp.logical_xor(mj, mk))
            K = jnp.where(keep, K, Kp)
            V = jnp.where(keep, V, Vp)
            jj //= 2
    o_ref[...] = V[:32, :]


def _tc_sort(rand_scores, interpret=False):
    s = rand_scores.reshape(_B, 64, 128)
    out = pl.pallas_call(
        _tc_sort_body,
        grid=(_B,),
        in_specs=[pl.BlockSpec((None, 64, 128), lambda b: (b, 0, 0))],
        out_specs=pl.BlockSpec((None, 32, 128), lambda b: (b, 0, 0)),
        out_shape=jax.ShapeDtypeStruct((_B, 32, 128), jnp.int32),
        interpret=interpret,
    )(s)
    return out.reshape(_B, _K)


def _gather_body(x_hbm, idx_hbm, out_hbm, idx_v, buf0, buf1, sem0, sem1):
    wid = lax.axis_index("s") * _NC + lax.axis_index("c")
    base = wid * _RPW
    pltpu.sync_copy(idx_hbm.at[wid], idx_v)

    bufs = (buf0, buf1)
    sems = (sem0, sem1)
    d0 = pltpu.async_copy(x_hbm.at[idx_v.at[0]], bufs[0], sems[0])
    descs = [d0, None]
    for c in range(_NCHUNK):
        descs[c % 2].wait()
        if c + 1 < _NCHUNK:
            descs[(c + 1) % 2] = pltpu.async_copy(
                x_hbm.at[idx_v.at[c + 1]], bufs[(c + 1) % 2], sems[(c + 1) % 2]
            )
        pltpu.sync_copy(bufs[c % 2], out_hbm.at[pl.ds(base + c * _CH, _CH)])


def _sc_gather(x_flat, idx_chunked):
    mesh = plsc.VectorSubcoreMesh(
        core_axis_name="c", subcore_axis_name="s", num_cores=_NC, num_subcores=_NS
    )
    return pl.kernel(
        _gather_body,
        out_type=jax.ShapeDtypeStruct((_ROWS, _D), jnp.float32),
        mesh=mesh,
        scratch_types=[
            pltpu.VMEM((_NCHUNK, _CH), jnp.int32),
            pltpu.VMEM((_CH, _D), jnp.float32),
            pltpu.VMEM((_CH, _D), jnp.float32),
            pltpu.SemaphoreType.DMA,
            pltpu.SemaphoreType.DMA,
        ],
    )(x_flat, idx_chunked)


def kernel(x, rand_scores):
    B, T, D = x.shape
    num_keep = _K
    token_indices_keep = _tc_sort(rand_scores)
    gidx = token_indices_keep + (jnp.arange(B, dtype=jnp.int32) * T)[:, None]
    gidx = gidx.reshape(_NW, _NCHUNK, _CH)
    out = _sc_gather(x.reshape(B * T, D), gidx)
    return (out.reshape(B, num_keep, D), token_indices_keep)


# speedup vs baseline: 1.6302x; 1.1562x over previous
import functools

import jax
import jax.numpy as jnp
from jax import lax
from jax.experimental import pallas as pl
from jax.experimental.pallas import tpu as pltpu
from jax.experimental.pallas import tpu_sc as plsc

_NC = 2
_NS = 16
_NW = _NC * _NS

_B, _T, _D = 4, 8192, 1024
_K = _T // 2
_ROWS = _B * _K
_RPW = _ROWS // _NW
_CH = 32
_NCHUNK = _RPW // _CH


_NV = _T // 16
_RADIX_BINS = 256 * 16


_NG = 4
_NVG = _NV // _NG


def _sort_body(scores_hbm, idx_hbm, sc_v, k0, v0, k1, v1, bins0, bins1, bins2,
               bins3, outb):
    wid = lax.axis_index("s") * _NC + lax.axis_index("c")
    binsg = (bins0, bins1, bins2, bins3)

    @pl.when(wid < _B)
    def _():
        r = wid
        pltpu.sync_copy(scores_hbm.at[r], sc_v)
        lane = lax.iota(jnp.int32, 16)
        ones = jnp.ones((16,), jnp.int32)
        zeros = jnp.zeros((16,), jnp.int32)

        def build(i, c):
            f = sc_v[pl.ds(i * 16, 16)]
            u = lax.bitcast_convert_type(f, jnp.int32)
            m = lax.shift_right_arithmetic(u, 31)
            key = u ^ (jnp.bitwise_not(m) & jnp.int32(0x7FFFFFFF))
            p = i * 16 + lane
            a = (p & (_NV - 1)) * 16 + lax.shift_right_logical(p, 9)
            plsc.store_scatter(k0, [a], key)
            plsc.store_scatter(v0, [a], p)
            return c

        lax.fori_loop(0, _NV, build, 0, unroll=4)

        bufs = ((k0, v0, k1, v1), (k1, v1, k0, v0))
        for pas in range(4):
            kin, vin, kout, vout = bufs[pas % 2]
            sh = 8 * pas

            def zb(i, c):
                for g in range(_NG):
                    binsg[g][pl.ds(i * 16, 16)] = zeros
                return c

            lax.fori_loop(0, _RADIX_BINS // 16, zb, 0, unroll=2)

            def hist(i, c):
                for g in range(_NG):
                    k = kin[pl.ds((g * _NVG + i) * 16, 16)]
                    d = (lax.shift_right_logical(k, sh) & jnp.int32(0xFF)) * 16 + lane
                    plsc.addupdate_scatter(binsg[g], [d], ones)
                return c

            lax.fori_loop(0, _NVG, hist, 0, unroll=2)

            def scan(i, run):
                sl = pl.ds(i * 16, 16)
                b = [binsg[g][sl] for g in range(_NG)]
                tot = b[0] + b[1] + b[2] + b[3]
                excl = (plsc.cumsum(tot) - tot) + run
                for g in range(_NG):
                    binsg[g][sl] = excl
                    excl = excl + b[g]
                return run + jnp.sum(tot)

            lax.fori_loop(0, _RADIX_BINS // 16, scan, jnp.int32(0))

            if pas < 3:

                def perm(i, c):
                    for g in range(_NG):
                        sl = pl.ds((g * _NVG + i) * 16, 16)
                        k = kin[sl]
                        val = vin[sl]
                        d = (lax.shift_right_logical(k, sh) & jnp.int32(0xFF)) * 16 + lane
                        q = plsc.load_gather(binsg[g], [d])
                        a = (q & (_NV - 1)) * 16 + lax.shift_right_logical(q, 9)
                        plsc.store_scatter(kout, [a], k)
                        plsc.store_scatter(vout, [a], val)
                        plsc.store_scatter(binsg[g], [d], q + 1)
                    return c

            else:
                def perm(i, c):
                    for g in range(_NG):
                        sl = pl.ds((g * _NVG + i) * 16, 16)
                        k = kin[sl]
                        val = vin[sl]
                        d = (lax.shift_right_logical(k, sh) & jnp.int32(0xFF)) * 16 + lane
                        q = plsc.load_gather(binsg[g], [d])
                        plsc.store_scatter(outb, [q], val, mask=q < _K)
                        plsc.store_scatter(binsg[g], [d], q + 1)
                    return c

            lax.fori_loop(0, _NVG, perm, 0, unroll=2)

        pltpu.sync_copy(outb, idx_hbm.at[r])


def _sc_sort(rand_scores):
    mesh = plsc.VectorSubcoreMesh(
        core_axis_name="c", subcore_axis_name="s", num_cores=_NC, num_subcores=_NS
    )
    return pl.kernel(
        _sort_body,
        out_type=jax.ShapeDtypeStruct((_B, _K), jnp.int32),
        mesh=mesh,
        compiler_params=pltpu.CompilerParams(
            use_tc_tiling_on_sc=False, needs_layout_passes=False
        ),
        scratch_types=[
            pltpu.VMEM((_T,), jnp.float32),
            pltpu.VMEM((_T,), jnp.int32),
            pltpu.VMEM((_T,), jnp.int32),
            pltpu.VMEM((_T,), jnp.int32),
            pltpu.VMEM((_T,), jnp.int32),
            pltpu.VMEM((_RADIX_BINS,), jnp.int32),
            pltpu.VMEM((_RADIX_BINS,), jnp.int32),
            pltpu.VMEM((_RADIX_BINS,), jnp.int32),
            pltpu.VMEM((_RADIX_BINS,), jnp.int32),
            pltpu.VMEM((_K,), jnp.int32),
        ],
    )(rand_scores)


_SR = _B * 64


def _tc_sort_body(s_ref, o_ref):
    f = s_ref[...]
    u = lax.bitcast_convert_type(f, jnp.int32)
    u = jnp.where(u == jnp.int32(-2147483648), jnp.int32(0), u)
    m = lax.shift_right_arithmetic(u, 31)
    key = u ^ (m & jnp.int32(0x7FFFFFFF))
    rsub = lax.broadcasted_iota(jnp.int32, (_SR, 128), 0) & 63
    lan = lax.broadcasted_iota(jnp.int32, (_SR, 128), 1)
    idx = rsub * 128 + lan
    K, V = key, idx
    for kk in [2 << t for t in range(13)]:
        mk = (
            (lan & kk) == 0 if kk < 128 else (rsub & (kk // 128)) == 0
        )
        jj = kk // 2
        while jj >= 1:
            if jj < 128:
                mj = (lan & jj) == 0
                Kp = jnp.where(mj, pltpu.roll(K, 128 - jj, 1), pltpu.roll(K, jj, 1))
                Vp = jnp.where(mj, pltpu.roll(V, 128 - jj, 1), pltpu.roll(V, jj, 1))
            else:
                js = jj // 128
                mj = (rsub & js) == 0
                Kp = jnp.where(mj, pltpu.roll(K, _SR - js, 0), pltpu.roll(K, js, 0))
                Vp = jnp.where(mj, pltpu.roll(V, _SR - js, 0), pltpu.roll(V, js, 0))
            self_first = (K > Kp) | ((K == Kp) & (V < Vp))
            keep = jnp.logical_xor(self_first, jnp.logical_xor(mj, mk))
            K = jnp.where(keep, K, Kp)
            V = jnp.where(keep, V, Vp)
            jj //= 2
    o_ref[...] = jnp.concatenate(
        [V[b * 64 : b * 64 + 32, :] for b in range(_B)], axis=0
    )


def _tc_sort(rand_scores, interpret=False):
    s = rand_scores.reshape(_SR, 128)
    out = pl.pallas_call(
        _tc_sort_body,
        out_shape=jax.ShapeDtypeStruct((_B * 32, 128), jnp.int32),
        interpret=interpret,
    )(s)
    return out.reshape(_B, _K)


def _gather_body(x_hbm, idx_hbm, out_hbm, idx_v, buf0, buf1, sem0, sem1):
    wid = lax.axis_index("s") * _NC + lax.axis_index("c")
    base = wid * _RPW
    pltpu.sync_copy(idx_hbm.at[wid], idx_v)

    bufs = (buf0, buf1)
    sems = (sem0, sem1)
    d0 = pltpu.async_copy(x_hbm.at[idx_v.at[0]], bufs[0], sems[0])
    descs = [d0, None]
    for c in range(_NCHUNK):
        descs[c % 2].wait()
        if c + 1 < _NCHUNK:
            descs[(c + 1) % 2] = pltpu.async_copy(
                x_hbm.at[idx_v.at[c + 1]], bufs[(c + 1) % 2], sems[(c + 1) % 2]
            )
        pltpu.sync_copy(bufs[c % 2], out_hbm.at[pl.ds(base + c * _CH, _CH)])


def _sc_gather(x_flat, idx_chunked):
    mesh = plsc.VectorSubcoreMesh(
        core_axis_name="c", subcore_axis_name="s", num_cores=_NC, num_subcores=_NS
    )
    return pl.kernel(
        _gather_body,
        out_type=jax.ShapeDtypeStruct((_ROWS, _D), jnp.float32),
        mesh=mesh,
        scratch_types=[
            pltpu.VMEM((_NCHUNK, _CH), jnp.int32),
            pltpu.VMEM((_CH, _D), jnp.float32),
            pltpu.VMEM((_CH, _D), jnp.float32),
            pltpu.SemaphoreType.DMA,
            pltpu.SemaphoreType.DMA,
        ],
    )(x_flat, idx_chunked)


def kernel(x, rand_scores):
    B, T, D = x.shape
    num_keep = _K
    token_indices_keep = _tc_sort(rand_scores)
    gidx = token_indices_keep + (jnp.arange(B, dtype=jnp.int32) * T)[:, None]
    gidx = gidx.reshape(_NW, _NCHUNK, _CH)
    out = _sc_gather(x.reshape(B * T, D), gidx)
    return (out.reshape(B, num_keep, D), token_indices_keep)


# speedup vs baseline: 1.7229x; 1.0569x over previous
import jax
import jax.numpy as jnp
from jax import lax
from jax.experimental import pallas as pl
from jax.experimental.pallas import tpu as pltpu
from jax.experimental.pallas import tpu_sc as plsc

_NC = 2
_NS = 16
_NW = _NC * _NS

_B, _T, _D = 4, 8192, 1024
_K = _T // 2
_ROWS = _B * _K
_RPW = _ROWS // _NW
_CH = 32
_NBUF = 3
_NCHUNK = _RPW // _CH

_SR = _B * 64


def _tc_sort_body(s_ref, loc_ref, glob_ref):
    f = s_ref[...]
    u = lax.bitcast_convert_type(f, jnp.int32)
    u = jnp.where(u == jnp.int32(-2147483648), jnp.int32(0), u)
    m = lax.shift_right_arithmetic(u, 31)
    key = u ^ (m & jnp.int32(0x7FFFFFFF))
    rfull = lax.broadcasted_iota(jnp.int32, (_SR, 128), 0)
    lan = lax.broadcasted_iota(jnp.int32, (_SR, 128), 1)
    rsub = rfull & 63
    idx = rfull * 128 + lan
    K, V = key, idx
    for kk in [2 << t for t in range(13)]:
        mk = (
            (lan & kk) == 0 if kk < 128 else (rsub & (kk // 128)) == 0
        )
        jj = kk // 2
        while jj >= 1:
            if jj < 128:
                mj = (lan & jj) == 0
                Kp = jnp.where(mj, pltpu.roll(K, 128 - jj, 1), pltpu.roll(K, jj, 1))
                Vp = jnp.where(mj, pltpu.roll(V, 128 - jj, 1), pltpu.roll(V, jj, 1))
            else:
                js = jj // 128
                mj = (rsub & js) == 0
                Kp = jnp.where(mj, pltpu.roll(K, _SR - js, 0), pltpu.roll(K, js, 0))
                Vp = jnp.where(mj, pltpu.roll(V, _SR - js, 0), pltpu.roll(V, js, 0))
            self_first = (K > Kp) | ((K == Kp) & (V < Vp))
            keep = jnp.logical_xor(self_first, jnp.logical_xor(mj, mk))
            K = jnp.where(keep, K, Kp)
            V = jnp.where(keep, V, Vp)
            jj //= 2
    vtop = jnp.concatenate(
        [V[b * 64 : b * 64 + 32, :] for b in range(_B)], axis=0
    )
    loc_ref[...] = vtop & jnp.int32(_T - 1)
    glob_ref[...] = vtop


def _tc_sort(rand_scores, interpret=False):
    s = rand_scores.reshape(_SR, 128)
    loc, glob = pl.pallas_call(
        _tc_sort_body,
        out_shape=[
            jax.ShapeDtypeStruct((_B * 32, 128), jnp.int32),
            jax.ShapeDtypeStruct((_B * 32, 128), jnp.int32),
        ],
        interpret=interpret,
    )(s)
    return loc.reshape(_B, _K), glob.reshape(_NW, _NCHUNK, _CH)


def _gather_body(x_hbm, idx_hbm, out_hbm, idx_v, buf0, buf1, buf2, sem0, sem1,
                 sem2):
    wid = lax.axis_index("s") * _NC + lax.axis_index("c")
    base = wid * _RPW
    pltpu.sync_copy(idx_hbm.at[wid], idx_v)

    bufs = (buf0, buf1, buf2)
    sems = (sem0, sem1, sem2)
    descs = [None] * _NBUF
    for c in range(_NBUF - 1):
        descs[c] = pltpu.async_copy(x_hbm.at[idx_v.at[c]], bufs[c], sems[c])
    for c in range(_NCHUNK):
        s = c % _NBUF
        if c + _NBUF - 1 < _NCHUNK:
            descs[(c + _NBUF - 1) % _NBUF] = pltpu.async_copy(
                x_hbm.at[idx_v.at[c + _NBUF - 1]],
                bufs[(c + _NBUF - 1) % _NBUF],
                sems[(c + _NBUF - 1) % _NBUF],
            )
        descs[s].wait()
        pltpu.sync_copy(bufs[s], out_hbm.at[pl.ds(base + c * _CH, _CH)])


def _sc_gather(x_flat, idx_chunked):
    mesh = plsc.VectorSubcoreMesh(
        core_axis_name="c", subcore_axis_name="s", num_cores=_NC, num_subcores=_NS
    )
    return pl.kernel(
        _gather_body,
        out_type=jax.ShapeDtypeStruct((_ROWS, _D), jnp.float32),
        mesh=mesh,
        scratch_types=[
            pltpu.VMEM((_NCHUNK, _CH), jnp.int32),
            pltpu.VMEM((_CH, _D), jnp.float32),
            pltpu.VMEM((_CH, _D), jnp.float32),
            pltpu.VMEM((_CH, _D), jnp.float32),
            pltpu.SemaphoreType.DMA,
            pltpu.SemaphoreType.DMA,
            pltpu.SemaphoreType.DMA,
        ],
    )(x_flat, idx_chunked)


def kernel(x, rand_scores):
    B, T, D = x.shape
    token_indices_keep, gidx = _tc_sort(rand_scores)
    out = _sc_gather(x.reshape(B * T, D), gidx)
    return (out.reshape(B, _K, D), token_indices_keep)
